# Initial kernel scaffold; baseline (speedup 1.0000x reference)
import jax
import jax.numpy as jnp
from jax.experimental import pallas as pl


def kernel(x, embeddings):
    raise NotImplementedError("write your pallas kernel here")


# speedup vs baseline: 2.6337x; 2.6337x over previous
import functools

import jax
import jax.numpy as jnp
from jax.experimental import pallas as pl
from jax.experimental.pallas import tpu as pltpu


def _vq_body(x_ref, cb_ref, out_ref, idx_ref, acc_ref, *, n_tokens, K):
    i = pl.program_id(0)
    xb = x_ref[0]
    cb = cb_ref[...]

    cbn = cb / jnp.maximum(
        jnp.sqrt(jnp.sum(cb * cb, axis=1, keepdims=True)), 1e-12)
    xn = xb / jnp.maximum(
        jnp.sqrt(jnp.sum(xb * xb, axis=0, keepdims=True)), 1e-12)

    scores = jax.lax.dot_general(
        cbn, xn, (((1,), (0,)), ((), ())),
        preferred_element_type=jnp.float32)

    maxv = jnp.max(scores, axis=0, keepdims=True)
    iota_k = jax.lax.broadcasted_iota(jnp.int32, scores.shape, 0)
    idx = jnp.min(jnp.where(scores == maxv, iota_k, K), axis=0,
                  keepdims=True)
    idx_ref[0] = idx

    onehot = (iota_k == idx).astype(jnp.bfloat16)
    out_ref[0] = jax.lax.dot_general(
        cbn.astype(jnp.bfloat16), onehot, (((0,), (0,)), ((), ())),
        preferred_element_type=jnp.float32)

    @pl.when(i == 0)
    def _():
        acc_ref[0, 0] = 0.0

    acc_ref[0, 0] += jnp.sum(maxv)

    @pl.when(i == pl.num_programs(0) - 1)
    def _():
        acc_ref[0, 0] = 2.0 - (2.0 / n_tokens) * acc_ref[0, 0]


def kernel(x, embeddings):
    B, D, L = x.shape
    K = embeddings.shape[0]
    out, idx, loss = pl.pallas_call(
        functools.partial(_vq_body, n_tokens=B * L, K=K),
        grid=(B,),
        in_specs=[
            pl.BlockSpec((1, D, L), lambda i: (i, 0, 0)),
            pl.BlockSpec((K, D), lambda i: (0, 0)),
        ],
        out_specs=[
            pl.BlockSpec((1, D, L), lambda i: (i, 0, 0)),
            pl.BlockSpec((1, 1, L), lambda i: (i, 0, 0)),
            pl.BlockSpec(memory_space=pltpu.SMEM, block_shape=(1, 1),
                         index_map=lambda i: (0, 0)),
        ],
        out_shape=[
            jax.ShapeDtypeStruct((B, D, L), jnp.float32),
            jax.ShapeDtypeStruct((B, 1, L), jnp.int32),
            jax.ShapeDtypeStruct((1, 1), jnp.float32),
        ],
    )(x, embeddings)
    del idx
    return out, loss[0, 0]


# speedup vs baseline: 3.1950x; 1.2131x over previous
import functools

import jax
import jax.numpy as jnp
from jax.experimental import pallas as pl
from jax.experimental.pallas import tpu as pltpu

_BB = 4


def _vq_body(x_ref, cb_ref, out_ref, idx_ref, acc_ref, cbn_ref, cbn16_ref,
             *, n_tokens, K):
    i = pl.program_id(0)

    @pl.when(i == 0)
    def _():
        cb = cb_ref[...]
        cbn = cb / jnp.maximum(
            jnp.sqrt(jnp.sum(cb * cb, axis=1, keepdims=True)), 1e-12)
        cbn_ref[...] = cbn
        cbn16_ref[...] = cbn.astype(jnp.bfloat16)
        acc_ref[0, 0] = 0.0

    cbn = cbn_ref[...]
    cbn16 = cbn16_ref[...]

    acc = 0.0
    for j in range(_BB):
        xb = x_ref[j]
        xn = xb / jnp.maximum(
            jnp.sqrt(jnp.sum(xb * xb, axis=0, keepdims=True)), 1e-12)

        scores = jax.lax.dot_general(
            cbn, xn, (((1,), (0,)), ((), ())),
            preferred_element_type=jnp.float32)

        maxv = jnp.max(scores, axis=0, keepdims=True)
        iota_k = jax.lax.broadcasted_iota(jnp.int32, scores.shape, 0)
        idx = jnp.min(jnp.where(scores == maxv, iota_k, K), axis=0,
                      keepdims=True)
        idx_ref[j, 0] = idx[0]

        onehot = (iota_k == idx).astype(jnp.bfloat16)
        out_ref[j] = jax.lax.dot_general(
            cbn16, onehot, (((0,), (0,)), ((), ())),
            preferred_element_type=jnp.float32)

        acc += jnp.sum(maxv)

    acc_ref[0, 0] += acc

    @pl.when(i == pl.num_programs(0) - 1)
    def _():
        acc_ref[0, 0] = 2.0 - (2.0 / n_tokens) * acc_ref[0, 0]


def kernel(x, embeddings):
    B, D, L = x.shape
    K = embeddings.shape[0]
    out, idx, loss = pl.pallas_call(
        functools.partial(_vq_body, n_tokens=B * L, K=K),
        grid=(B // _BB,),
        in_specs=[
            pl.BlockSpec((_BB, D, L), lambda i: (i, 0, 0)),
            pl.BlockSpec((K, D), lambda i: (0, 0)),
        ],
        out_specs=[
            pl.BlockSpec((_BB, D, L), lambda i: (i, 0, 0)),
            pl.BlockSpec((_BB, 1, L), lambda i: (i, 0, 0)),
            pl.BlockSpec(memory_space=pltpu.SMEM, block_shape=(1, 1),
                         index_map=lambda i: (0, 0)),
        ],
        out_shape=[
            jax.ShapeDtypeStruct((B, D, L), jnp.float32),
            jax.ShapeDtypeStruct((B, 1, L), jnp.int32),
            jax.ShapeDtypeStruct((1, 1), jnp.float32),
        ],
        scratch_shapes=[
            pltpu.VMEM((K, D), jnp.float32),
            pltpu.VMEM((K, D), jnp.bfloat16),
        ],
    )(x, embeddings)
    del idx
    return out, loss[0, 0]
